# Initial kernel scaffold; baseline (speedup 1.0000x reference)
import jax
import jax.numpy as jnp
from jax.experimental import pallas as pl


def kernel(edge_feature, emb0, emb1, emb2, emb3, W_amount, b_amount, W_time, b_time):
    raise NotImplementedError("write your pallas kernel here")


# speedup vs baseline: 4.8762x; 4.8762x over previous
import functools

import jax
import jax.numpy as jnp
from jax import lax
from jax.experimental import pallas as pl
from jax.experimental.pallas import tpu as pltpu
from jax.experimental.pallas import tpu_sc as plsc

INT_DIM = 32
EMB_DIM = 128
OUT_DIM = 256
LANES = 16
B = 400
BT = 1024


def _sc_embed(cols, tabs, E):
    info = plsc.get_sparse_core_info()
    nw = info.num_cores * info.num_subcores
    rows_per_w = E // nw
    assert rows_per_w % B == 0
    nblocks = rows_per_w // B
    assert nblocks % 2 == 0
    ngroups = B // LANES

    mesh = plsc.VectorSubcoreMesh(core_axis_name="c", subcore_axis_name="s")

    @functools.partial(
        pl.kernel,
        mesh=mesh,
        out_type=jax.ShapeDtypeStruct((E, OUT_DIM), jnp.float32),
        compiler_params=pltpu.CompilerParams(
            use_tc_tiling_on_sc=False, needs_layout_passes=False
        ),
        scratch_types=[
            [pltpu.VMEM(t.shape, jnp.float32) for t in tabs],
            [pltpu.VMEM((4, B), jnp.float32) for _ in range(2)],
            pltpu.VMEM((4, B), jnp.int32),
            [pltpu.VMEM((B, EMB_DIM), jnp.float32) for _ in range(2)],
            [pltpu.SemaphoreType.DMA for _ in range(2)],
            [pltpu.SemaphoreType.DMA for _ in range(2)],
        ],
    )
    def k(c0, c1, c2, c3, e0, e1, e2, e3, out_hbm,
          tabs_v, efs, idx_v, embs, ssems, wsems):
        cols_hbm = (c0, c1, c2, c3)
        tabs_hbm = (e0, e1, e2, e3)
        wid = lax.axis_index("s") * info.num_cores + lax.axis_index("c")
        w0 = wid * rows_per_w

        for th, tv in zip(tabs_hbm, tabs_v):
            pltpu.sync_copy(th, tv)
        iota = lax.iota(jnp.int32, LANES)
        iota_hi = iota + LANES
        cc = [jnp.full((LANES,), c, jnp.int32) for c in range(4)]

        def fire_stage(blk, s):
            base = w0 + blk * B
            for c in range(4):
                pltpu.async_copy(cols_hbm[c].at[pl.ds(base, B)], efs[s].at[c], ssems[s])

        fire_stage(0, 0)
        fire_stage(1, 1)

        def outer(bb, carry):
            for s in range(2):
                blk = bb * 2 + s
                base = w0 + blk * B

                for c in range(4):
                    pltpu.make_async_copy(
                        cols_hbm[c].at[pl.ds(base, B)], efs[s].at[c], ssems[s]
                    ).wait()

                @pl.when(bb > 0)
                def _drain_write():
                    pltpu.make_async_copy(
                        embs[s], out_hbm.at[pl.ds(base, B), pl.ds(0, EMB_DIM)], wsems[s]
                    ).wait()

                def grp(g, c2):
                    o = g * LANES
                    for c in range(4):
                        idx_v[c, pl.ds(o, LANES)] = efs[s][c, pl.ds(o, LANES)].astype(jnp.int32)
                    for r in range(LANES):
                        row = o + r
                        sel = jnp.full((LANES,), row, jnp.int32)
                        for c in range(4):
                            iv = plsc.load_gather(idx_v, [cc[c], sel])
                            lo = plsc.load_gather(tabs_v[c], [iv, iota])
                            hi = plsc.load_gather(tabs_v[c], [iv, iota_hi])
                            embs[s][row, pl.ds(INT_DIM * c, LANES)] = lo
                            embs[s][row, pl.ds(INT_DIM * c + LANES, LANES)] = hi
                    return c2

                lax.fori_loop(0, ngroups, grp, 0)

                pltpu.async_copy(
                    embs[s], out_hbm.at[pl.ds(base, B), pl.ds(0, EMB_DIM)], wsems[s]
                )

                @pl.when(blk + 2 < nblocks)
                def _prefetch():
                    fire_stage(blk + 2, s)
            return carry

        lax.fori_loop(0, nblocks // 2, outer, 0)

        for s in range(2):
            pltpu.make_async_copy(
                embs[s], out_hbm.at[pl.ds(0, B), pl.ds(0, EMB_DIM)], wsems[s]
            ).wait()

    return k(*cols, *tabs)


def _tc_linear(xall, amt, tim, wa, ba, wt, bt, E):
    def body(x_any, a_ref, t_ref, wa_ref, ba_ref, wt_ref, bt_ref, o_ref):
        o_ref[:, 0:64] = a_ref[:, :] * wa_ref[:, :] + ba_ref[:, :]
        o_ref[:, 64:128] = t_ref[:, :] * wt_ref[:, :] + bt_ref[:, :]

    return pl.pallas_call(
        body,
        grid=(E // BT,),
        in_specs=[
            pl.BlockSpec(memory_space=pl.ANY),
            pl.BlockSpec((BT, 1), lambda i: (i, 0)),
            pl.BlockSpec((BT, 1), lambda i: (i, 0)),
            pl.BlockSpec((1, 64), lambda i: (0, 0)),
            pl.BlockSpec((1, 64), lambda i: (0, 0)),
            pl.BlockSpec((1, 64), lambda i: (0, 0)),
            pl.BlockSpec((1, 64), lambda i: (0, 0)),
        ],
        out_specs=pl.BlockSpec((BT, 128), lambda i: (i, 1)),
        out_shape=jax.ShapeDtypeStruct((E, OUT_DIM), jnp.float32),
        input_output_aliases={0: 0},
    )(xall, amt, tim, wa, ba, wt, bt)


def kernel(edge_feature, emb0, emb1, emb2, emb3, W_amount, b_amount, W_time, b_time):
    E = edge_feature.shape[0]
    colsT = edge_feature.T
    cols = [colsT[i] for i in range(4)]
    x = _sc_embed(cols, (emb0, emb1, emb2, emb3), E)
    return _tc_linear(
        x,
        edge_feature[:, 4:5], edge_feature[:, 5:6],
        W_amount, b_amount.reshape(1, 64),
        W_time, b_time.reshape(1, 64),
        E,
    )
